# baseline (speedup 1.0000x reference)
import functools

import jax
import jax.numpy as jnp
from jax import lax
from jax.experimental import pallas as pl
from jax.experimental.pallas import tpu as pltpu
from jax.experimental.pallas import tpu_sc as plsc

N = 10000
E = 320000
D = 128
NG = 64
DIM_EMB = 128
DIM_OUT = 16

CHUNK = 128
NUM_CHUNKS = E // CHUNK
TILES = 16
CHUNKS_PER_TILE = (NUM_CHUNKS + TILES - 1) // TILES
SLICE = 624
TAIL = N - TILES * SLICE
DEGW = 16


def _sc_aggregate(x0, ei0, x1, ei1, z128, z16, ones16):
    mesh = plsc.VectorSubcoreMesh(core_axis_name="c", subcore_axis_name="s")

    @functools.partial(
        pl.kernel,
        mesh=mesh,
        compiler_params=pltpu.CompilerParams(use_tc_tiling_on_sc=False),
        out_type=[
            jax.ShapeDtypeStruct((N, D), jnp.float32),
            jax.ShapeDtypeStruct((N, DEGW), jnp.float32),
            jax.ShapeDtypeStruct((N, D), jnp.float32),
            jax.ShapeDtypeStruct((N, DEGW), jnp.float32),
        ],
        scratch_types=[
            pltpu.VMEM((2, CHUNK), jnp.int32),
            pltpu.VMEM((2, CHUNK), jnp.int32),
            pltpu.VMEM((CHUNK, D), jnp.float32),
            pltpu.VMEM((CHUNK, D), jnp.float32),
            pltpu.VMEM((CHUNK, DEGW), jnp.float32),
            pltpu.VMEM_SHARED((N, D), jnp.float32),
            pltpu.VMEM_SHARED((N, DEGW), jnp.float32),
            pltpu.SemaphoreType.DMA,
            pltpu.SemaphoreType.DMA,
            pltpu.SemaphoreType.DMA,
            pltpu.SemaphoreType.DMA,
            pltpu.SemaphoreType.DMA,
            pltpu.SemaphoreType.DMA,
        ],
    )
    def sc_kernel(x0_h, ei0_h, x1_h, ei1_h,
                  z128_h, z16_h, ones_h,
                  agg0_h, deg0_h, agg1_h, deg1_h,
                  idx_a, idx_b, rows_va, rows_vb, ones_v,
                  acc_sh, deg_sh, sem, sem_sa, sem_sb, sem_ib, sem_g, sem_gb):
        cid = lax.axis_index("c")
        sid = lax.axis_index("s")
        row0 = sid * SLICE
        pieces = []
        off = 0
        while off < SLICE:
            sz = min(CHUNK, SLICE - off)
            pieces.append((off, sz))
            off += sz

        def run_branch(x_h, ei_h, agg_h, deg_h):
            pltpu.sync_copy(z128_h.at[pl.ds(0, CHUNK)], rows_va)
            pltpu.sync_copy(z16_h.at[pl.ds(0, CHUNK)], ones_v)
            for (o, sz) in pieces:
                pltpu.sync_copy(rows_va.at[pl.ds(0, sz)],
                                acc_sh.at[pl.ds(row0 + o, sz)])
                pltpu.sync_copy(ones_v.at[pl.ds(0, sz)],
                                deg_sh.at[pl.ds(row0 + o, sz)])

            @pl.when(sid == TILES - 1)
            def _():
                pltpu.sync_copy(rows_va.at[pl.ds(0, TAIL)],
                                acc_sh.at[pl.ds(TILES * SLICE, TAIL)])
                pltpu.sync_copy(ones_v.at[pl.ds(0, TAIL)],
                                deg_sh.at[pl.ds(TILES * SLICE, TAIL)])

            pltpu.sync_copy(ones_h, ones_v)
            plsc.subcore_barrier()

            def fetch_idx(g, buf, fsem):
                return pltpu.async_copy(
                    ei_h.at[:, pl.ds(g * CHUNK, CHUNK)], buf, fsem)

            def start_gather(buf, rows, gsem):
                return pltpu.async_copy(x_h.at[buf.at[0]], rows, gsem)

            def wait_gather(buf, rows, gsem):
                pltpu.make_async_copy(x_h.at[buf.at[0]], rows, gsem).wait()

            def issue_scatter(buf, rows, ssem):
                pltpu.async_copy(rows, acc_sh.at[buf.at[1]], ssem, add=True)
                pltpu.async_copy(ones_v, deg_sh.at[buf.at[1]], ssem,
                                 add=True)

            def drain_scatter(buf, rows, ssem):
                pltpu.make_async_copy(rows, acc_sh.at[buf.at[1]], ssem).wait()
                pltpu.make_async_copy(ones_v, deg_sh.at[buf.at[1]],
                                      ssem).wait()

            def wait_idx(g, buf, fsem):
                pltpu.make_async_copy(
                    ei_h.at[:, pl.ds(g * CHUNK, CHUNK)], buf, fsem).wait()

            @pl.when(sid < NUM_CHUNKS)
            def _():
                fetch_idx(sid, idx_a, sem)
                wait_idx(sid, idx_a, sem)
                start_gather(idx_a, rows_va, sem_g)

            def body(i, carry):
                ga = (2 * i) * TILES + sid
                gb = ga + TILES
                ga2 = ga + 2 * TILES
                gb_prev = ga - TILES

                @pl.when((gb_prev >= 0) & (gb_prev < NUM_CHUNKS))
                def _():
                    drain_scatter(idx_b, rows_vb, sem_sb)

                @pl.when(ga < NUM_CHUNKS)
                def _():
                    wait_gather(idx_a, rows_va, sem_g)

                    @pl.when(gb < NUM_CHUNKS)
                    def _():
                        fetch_idx(gb, idx_b, sem_ib)

                    issue_scatter(idx_a, rows_va, sem_sa)

                    @pl.when(gb < NUM_CHUNKS)
                    def _():
                        wait_idx(gb, idx_b, sem_ib)
                        start_gather(idx_b, rows_vb, sem_gb)

                @pl.when(gb < NUM_CHUNKS)
                def _():
                    wait_gather(idx_b, rows_vb, sem_gb)
                    drain_scatter(idx_a, rows_va, sem_sa)

                    @pl.when(ga2 < NUM_CHUNKS)
                    def _():
                        fetch_idx(ga2, idx_a, sem)

                    issue_scatter(idx_b, rows_vb, sem_sb)

                    @pl.when(ga2 < NUM_CHUNKS)
                    def _():
                        wait_idx(ga2, idx_a, sem)
                        start_gather(idx_a, rows_va, sem_g)

                return carry

            lax.fori_loop(0, (CHUNKS_PER_TILE + 1) // 2, body, 0)
            ga_last = (CHUNKS_PER_TILE - 1) * TILES + sid

            @pl.when(ga_last < NUM_CHUNKS)
            def _():
                drain_scatter(idx_a, rows_va, sem_sa)
            plsc.subcore_barrier()
            for (o, sz) in pieces:
                pltpu.sync_copy(acc_sh.at[pl.ds(row0 + o, sz)],
                                rows_va.at[pl.ds(0, sz)])
                pltpu.sync_copy(rows_va.at[pl.ds(0, sz)],
                                agg_h.at[pl.ds(row0 + o, sz)])
                pltpu.sync_copy(deg_sh.at[pl.ds(row0 + o, sz)],
                                ones_v.at[pl.ds(0, sz)])
                pltpu.sync_copy(ones_v.at[pl.ds(0, sz)],
                                deg_h.at[pl.ds(row0 + o, sz)])

            @pl.when(sid == TILES - 1)
            def _():
                pltpu.sync_copy(acc_sh.at[pl.ds(TILES * SLICE, TAIL)],
                                rows_va.at[pl.ds(0, TAIL)])
                pltpu.sync_copy(rows_va.at[pl.ds(0, TAIL)],
                                agg_h.at[pl.ds(TILES * SLICE, TAIL)])
                pltpu.sync_copy(deg_sh.at[pl.ds(TILES * SLICE, TAIL)],
                                ones_v.at[pl.ds(0, TAIL)])
                pltpu.sync_copy(ones_v.at[pl.ds(0, TAIL)],
                                deg_h.at[pl.ds(TILES * SLICE, TAIL)])

        @pl.when(cid == 0)
        def _():
            run_branch(x0_h, ei0_h, agg0_h, deg0_h)

        @pl.when(cid == 1)
        def _():
            run_branch(x1_h, ei1_h, agg1_h, deg1_h)

    return sc_kernel(x0, ei0, x1, ei1, z128, z16, ones16)


BLK = 1000
NUM_BLK = N // BLK


def _tc_body(agg0_r, deg0_r, b0m_r, agg1_r, deg1_r, b1m_r,
             W0_r, bb0_r, W1_r, bb1_r, f1a_r, f1b_r, f1bias_r, f2w_r, f2b_r,
             out_r, h_r, acc0, cnt0, acc1, cnt1):
    k = pl.program_id(0)

    @pl.when(k == 0)
    def _():
        acc0[...] = jnp.zeros_like(acc0)
        cnt0[...] = jnp.zeros_like(cnt0)
        acc1[...] = jnp.zeros_like(acc1)
        cnt1[...] = jnp.zeros_like(cnt1)

    ones_col = jnp.ones((BLK, 1), jnp.float32)
    gids = lax.broadcasted_iota(jnp.int32, (BLK, NG), 1)

    def branch(agg_r, deg_r, bm_r, W_r, bias_r, acc, cnt):
        deg = jnp.max(deg_r[...], axis=1, keepdims=True)
        inv = 1.0 / jnp.maximum(deg, 1.0)
        h = jnp.maximum(
            jnp.dot(agg_r[...] * inv, W_r[...],
                    preferred_element_type=jnp.float32) + bias_r[...],
            0.0)
        m = (bm_r[...] == gids).astype(jnp.float32)
        acc[...] += lax.dot_general(m, h, (((0,), (0,)), ((), ())),
                                    preferred_element_type=jnp.float32)
        cnt[...] += lax.dot_general(m, ones_col, (((0,), (0,)), ((), ())),
                                    preferred_element_type=jnp.float32)

    branch(agg0_r, deg0_r, b0m_r, W0_r, bb0_r, acc0, cnt0)
    branch(agg1_r, deg1_r, b1m_r, W1_r, bb1_r, acc1, cnt1)

    @pl.when(k == NUM_BLK - 1)
    def _():
        g0 = acc0[...] / jnp.maximum(cnt0[...], 1.0)
        g1 = acc1[...] / jnp.maximum(cnt1[...], 1.0)
        hh = jnp.maximum(
            jnp.dot(g0, f1a_r[...], preferred_element_type=jnp.float32)
            + jnp.dot(g1, f1b_r[...], preferred_element_type=jnp.float32)
            + f1bias_r[...], 0.0)
        h_r[...] = hh
        out_r[...] = jnp.dot(hh, f2w_r[...],
                             preferred_element_type=jnp.float32) + f2b_r[...]


def _tc_head(agg0, deg0, batch0, agg1, deg1, batch1,
             W0, b0, W1, b1, f1a, f1b, f1bias, f2w, f2b):
    row_spec = lambda shape: pl.BlockSpec((BLK,) + shape[1:],
                                          lambda k: (k,) + (0,) * (len(shape) - 1))
    full = lambda shape: pl.BlockSpec(shape, lambda k: (0,) * len(shape))
    return pl.pallas_call(
        _tc_body,
        grid=(NUM_BLK,),
        in_specs=[
            row_spec((N, D)), row_spec((N, DEGW)), row_spec((N, 1)),
            row_spec((N, D)), row_spec((N, DEGW)), row_spec((N, 1)),
            full((D, DIM_EMB)), full((1, DIM_EMB)),
            full((D, DIM_EMB)), full((1, DIM_EMB)),
            full((DIM_EMB, 64)), full((DIM_EMB, 64)), full((1, 64)),
            full((64, DIM_OUT)), full((1, DIM_OUT)),
        ],
        out_specs=[full((NG, DIM_OUT)), full((NG, 64))],
        out_shape=[jax.ShapeDtypeStruct((NG, DIM_OUT), jnp.float32),
                   jax.ShapeDtypeStruct((NG, 64), jnp.float32)],
        scratch_shapes=[
            pltpu.VMEM((NG, DIM_EMB), jnp.float32),
            pltpu.VMEM((NG, 1), jnp.float32),
            pltpu.VMEM((NG, DIM_EMB), jnp.float32),
            pltpu.VMEM((NG, 1), jnp.float32),
        ],
    )(agg0, deg0, batch0, agg1, deg1, batch1,
      W0, b0, W1, b1, f1a, f1b, f1bias, f2w, f2b)


def kernel(x0, edge_index0, batch0, x1, edge_index1, batch1,
           W0, b0, W1, b1, fc1_W, fc1_b, fc2_W, fc2_b):
    z128 = jnp.zeros((SLICE, D), jnp.float32)
    z16 = jnp.zeros((SLICE, DEGW), jnp.float32)
    ones16 = jnp.ones((CHUNK, DEGW), jnp.float32)

    agg0, deg0, agg1, deg1 = _sc_aggregate(
        x0, edge_index0, x1, edge_index1, z128, z16, ones16)

    out, h = _tc_head(
        agg0, deg0, batch0[:, None], agg1, deg1, batch1[:, None],
        W0, b0[None, :], W1, b1[None, :],
        fc1_W[:DIM_EMB], fc1_W[DIM_EMB:], fc1_b[None, :],
        fc2_W, fc2_b[None, :])
    return (out, h)


# speedup vs baseline: 1.2239x; 1.2239x over previous
import functools

import jax
import jax.numpy as jnp
from jax import lax
from jax.experimental import pallas as pl
from jax.experimental.pallas import tpu as pltpu
from jax.experimental.pallas import tpu_sc as plsc

N = 10000
E = 320000
D = 128
NG = 64
DIM_EMB = 128
DIM_OUT = 16

CHUNK = 128
NUM_CHUNKS = E // CHUNK
TILES = 16
CHUNKS_PER_TILE = (NUM_CHUNKS + TILES - 1) // TILES
SLICE = 624
TAIL = N - TILES * SLICE
DEGW = 16


def _sc_aggregate(x0, ei0, x1, ei1, z128, z16, ones16):
    mesh = plsc.VectorSubcoreMesh(core_axis_name="c", subcore_axis_name="s")

    @functools.partial(
        pl.kernel,
        mesh=mesh,
        compiler_params=pltpu.CompilerParams(use_tc_tiling_on_sc=False),
        out_type=[
            jax.ShapeDtypeStruct((N, D), jnp.float32),
            jax.ShapeDtypeStruct((N, DEGW), jnp.float32),
            jax.ShapeDtypeStruct((N, D), jnp.float32),
            jax.ShapeDtypeStruct((N, DEGW), jnp.float32),
        ],
        scratch_types=[
            pltpu.VMEM((2, CHUNK), jnp.int32),
            pltpu.VMEM((2, CHUNK), jnp.int32),
            pltpu.VMEM((CHUNK, D), jnp.float32),
            pltpu.VMEM((CHUNK, D), jnp.float32),
            pltpu.VMEM((CHUNK, DEGW), jnp.float32),
            pltpu.VMEM_SHARED((N, D), jnp.float32),
            pltpu.VMEM_SHARED((N, DEGW), jnp.float32),
            pltpu.SemaphoreType.DMA,
            pltpu.SemaphoreType.DMA,
            pltpu.SemaphoreType.DMA,
            pltpu.SemaphoreType.DMA,
            pltpu.SemaphoreType.DMA,
        ],
    )
    def sc_kernel(x0_h, ei0_h, x1_h, ei1_h,
                  z128_h, z16_h, ones_h,
                  agg0_h, deg0_h, agg1_h, deg1_h,
                  idx_a, idx_b, rows_va, rows_vb, ones_v,
                  acc_sh, deg_sh, sem, sem_s, sem_ib, sem_g, sem_gb):
        cid = lax.axis_index("c")
        sid = lax.axis_index("s")
        row0 = sid * SLICE
        pieces = []
        off = 0
        while off < SLICE:
            sz = min(CHUNK, SLICE - off)
            pieces.append((off, sz))
            off += sz

        def run_branch(x_h, ei_h, agg_h, deg_h):
            pltpu.sync_copy(z128_h.at[pl.ds(0, CHUNK)], rows_va)
            pltpu.sync_copy(z16_h.at[pl.ds(0, CHUNK)], ones_v)
            for (o, sz) in pieces:
                pltpu.sync_copy(rows_va.at[pl.ds(0, sz)],
                                acc_sh.at[pl.ds(row0 + o, sz)])
                pltpu.sync_copy(ones_v.at[pl.ds(0, sz)],
                                deg_sh.at[pl.ds(row0 + o, sz)])

            @pl.when(sid == TILES - 1)
            def _():
                pltpu.sync_copy(rows_va.at[pl.ds(0, TAIL)],
                                acc_sh.at[pl.ds(TILES * SLICE, TAIL)])
                pltpu.sync_copy(ones_v.at[pl.ds(0, TAIL)],
                                deg_sh.at[pl.ds(TILES * SLICE, TAIL)])

            pltpu.sync_copy(ones_h, ones_v)
            plsc.subcore_barrier()

            def fetch_idx(g, buf, fsem):
                return pltpu.async_copy(
                    ei_h.at[:, pl.ds(g * CHUNK, CHUNK)], buf, fsem)

            def start_gather(buf, rows, gsem):
                return pltpu.async_copy(x_h.at[buf.at[0]], rows, gsem)

            def wait_gather(buf, rows, gsem):
                pltpu.make_async_copy(x_h.at[buf.at[0]], rows, gsem).wait()

            def scatter(buf, rows):
                s0 = pltpu.async_copy(rows, acc_sh.at[buf.at[1]],
                                      sem_s, add=True)
                s1 = pltpu.async_copy(ones_v, deg_sh.at[buf.at[1]],
                                      sem_s, add=True)
                s0.wait()
                s1.wait()

            def wait_idx(g, buf, fsem):
                pltpu.make_async_copy(
                    ei_h.at[:, pl.ds(g * CHUNK, CHUNK)], buf, fsem).wait()

            @pl.when(sid < NUM_CHUNKS)
            def _():
                fetch_idx(sid, idx_a, sem)

            @pl.when(sid + TILES < NUM_CHUNKS)
            def _():
                fetch_idx(sid + TILES, idx_b, sem_ib)

            @pl.when(sid < NUM_CHUNKS)
            def _():
                wait_idx(sid, idx_a, sem)
                start_gather(idx_a, rows_va, sem_g)

            def body(i, carry):
                ga = (2 * i) * TILES + sid
                gb = ga + TILES
                ga2 = ga + 2 * TILES
                gb2 = ga + 3 * TILES

                @pl.when(ga < NUM_CHUNKS)
                def _():
                    wait_gather(idx_a, rows_va, sem_g)

                    @pl.when(gb < NUM_CHUNKS)
                    def _():
                        wait_idx(gb, idx_b, sem_ib)
                        start_gather(idx_b, rows_vb, sem_gb)

                    scatter(idx_a, rows_va)

                @pl.when(ga2 < NUM_CHUNKS)
                def _():
                    fetch_idx(ga2, idx_a, sem)

                @pl.when(gb < NUM_CHUNKS)
                def _():
                    wait_gather(idx_b, rows_vb, sem_gb)

                    @pl.when(ga2 < NUM_CHUNKS)
                    def _():
                        wait_idx(ga2, idx_a, sem)
                        start_gather(idx_a, rows_va, sem_g)

                    scatter(idx_b, rows_vb)

                @pl.when(gb2 < NUM_CHUNKS)
                def _():
                    fetch_idx(gb2, idx_b, sem_ib)

                return carry

            lax.fori_loop(0, (CHUNKS_PER_TILE + 1) // 2, body, 0)
            plsc.subcore_barrier()
            for (o, sz) in pieces:
                pltpu.sync_copy(acc_sh.at[pl.ds(row0 + o, sz)],
                                rows_va.at[pl.ds(0, sz)])
                pltpu.sync_copy(rows_va.at[pl.ds(0, sz)],
                                agg_h.at[pl.ds(row0 + o, sz)])
                pltpu.sync_copy(deg_sh.at[pl.ds(row0 + o, sz)],
                                ones_v.at[pl.ds(0, sz)])
                pltpu.sync_copy(ones_v.at[pl.ds(0, sz)],
                                deg_h.at[pl.ds(row0 + o, sz)])

            @pl.when(sid == TILES - 1)
            def _():
                pltpu.sync_copy(acc_sh.at[pl.ds(TILES * SLICE, TAIL)],
                                rows_va.at[pl.ds(0, TAIL)])
                pltpu.sync_copy(rows_va.at[pl.ds(0, TAIL)],
                                agg_h.at[pl.ds(TILES * SLICE, TAIL)])
                pltpu.sync_copy(deg_sh.at[pl.ds(TILES * SLICE, TAIL)],
                                ones_v.at[pl.ds(0, TAIL)])
                pltpu.sync_copy(ones_v.at[pl.ds(0, TAIL)],
                                deg_h.at[pl.ds(TILES * SLICE, TAIL)])

        @pl.when(cid == 0)
        def _():
            run_branch(x0_h, ei0_h, agg0_h, deg0_h)

        @pl.when(cid == 1)
        def _():
            run_branch(x1_h, ei1_h, agg1_h, deg1_h)

    return sc_kernel(x0, ei0, x1, ei1, z128, z16, ones16)


BLK = 1000
NUM_BLK = N // BLK


def _tc_body(agg0_r, deg0_r, b0m_r, agg1_r, deg1_r, b1m_r,
             W0_r, bb0_r, W1_r, bb1_r, f1a_r, f1b_r, f1bias_r, f2w_r, f2b_r,
             out_r, h_r, acc0, cnt0, acc1, cnt1):
    k = pl.program_id(0)

    @pl.when(k == 0)
    def _():
        acc0[...] = jnp.zeros_like(acc0)
        cnt0[...] = jnp.zeros_like(cnt0)
        acc1[...] = jnp.zeros_like(acc1)
        cnt1[...] = jnp.zeros_like(cnt1)

    ones_col = jnp.ones((BLK, 1), jnp.float32)
    gids = lax.broadcasted_iota(jnp.int32, (BLK, NG), 1)

    def branch(agg_r, deg_r, bm_r, W_r, bias_r, acc, cnt):
        deg = jnp.max(deg_r[...], axis=1, keepdims=True)
        inv = 1.0 / jnp.maximum(deg, 1.0)
        h = jnp.maximum(
            jnp.dot(agg_r[...] * inv, W_r[...],
                    preferred_element_type=jnp.float32) + bias_r[...],
            0.0)
        m = (bm_r[...] == gids).astype(jnp.float32)
        acc[...] += lax.dot_general(m, h, (((0,), (0,)), ((), ())),
                                    preferred_element_type=jnp.float32)
        cnt[...] += lax.dot_general(m, ones_col, (((0,), (0,)), ((), ())),
                                    preferred_element_type=jnp.float32)

    branch(agg0_r, deg0_r, b0m_r, W0_r, bb0_r, acc0, cnt0)
    branch(agg1_r, deg1_r, b1m_r, W1_r, bb1_r, acc1, cnt1)

    @pl.when(k == NUM_BLK - 1)
    def _():
        g0 = acc0[...] / jnp.maximum(cnt0[...], 1.0)
        g1 = acc1[...] / jnp.maximum(cnt1[...], 1.0)
        hh = jnp.maximum(
            jnp.dot(g0, f1a_r[...], preferred_element_type=jnp.float32)
            + jnp.dot(g1, f1b_r[...], preferred_element_type=jnp.float32)
            + f1bias_r[...], 0.0)
        h_r[...] = hh
        out_r[...] = jnp.dot(hh, f2w_r[...],
                             preferred_element_type=jnp.float32) + f2b_r[...]


def _tc_head(agg0, deg0, batch0, agg1, deg1, batch1,
             W0, b0, W1, b1, f1a, f1b, f1bias, f2w, f2b):
    row_spec = lambda shape: pl.BlockSpec((BLK,) + shape[1:],
                                          lambda k: (k,) + (0,) * (len(shape) - 1))
    full = lambda shape: pl.BlockSpec(shape, lambda k: (0,) * len(shape))
    return pl.pallas_call(
        _tc_body,
        grid=(NUM_BLK,),
        in_specs=[
            row_spec((N, D)), row_spec((N, DEGW)), row_spec((N, 1)),
            row_spec((N, D)), row_spec((N, DEGW)), row_spec((N, 1)),
            full((D, DIM_EMB)), full((1, DIM_EMB)),
            full((D, DIM_EMB)), full((1, DIM_EMB)),
            full((DIM_EMB, 64)), full((DIM_EMB, 64)), full((1, 64)),
            full((64, DIM_OUT)), full((1, DIM_OUT)),
        ],
        out_specs=[full((NG, DIM_OUT)), full((NG, 64))],
        out_shape=[jax.ShapeDtypeStruct((NG, DIM_OUT), jnp.float32),
                   jax.ShapeDtypeStruct((NG, 64), jnp.float32)],
        scratch_shapes=[
            pltpu.VMEM((NG, DIM_EMB), jnp.float32),
            pltpu.VMEM((NG, 1), jnp.float32),
            pltpu.VMEM((NG, DIM_EMB), jnp.float32),
            pltpu.VMEM((NG, 1), jnp.float32),
        ],
    )(agg0, deg0, batch0, agg1, deg1, batch1,
      W0, b0, W1, b1, f1a, f1b, f1bias, f2w, f2b)


def kernel(x0, edge_index0, batch0, x1, edge_index1, batch1,
           W0, b0, W1, b1, fc1_W, fc1_b, fc2_W, fc2_b):
    z128 = jnp.zeros((SLICE, D), jnp.float32)
    z16 = jnp.zeros((SLICE, DEGW), jnp.float32)
    ones16 = jnp.ones((CHUNK, DEGW), jnp.float32)

    agg0, deg0, agg1, deg1 = _sc_aggregate(
        x0, edge_index0, x1, edge_index1, z128, z16, ones16)

    out, h = _tc_head(
        agg0, deg0, batch0[:, None], agg1, deg1, batch1[:, None],
        W0, b0[None, :], W1, b1[None, :],
        fc1_W[:DIM_EMB], fc1_W[DIM_EMB:], fc1_b[None, :],
        fc2_W, fc2_b[None, :])
    return (out, h)
